# baseline (speedup 1.0000x reference)
import jax
import jax.numpy as jnp
from jax import lax
from jax.experimental import pallas as pl
from jax.experimental.pallas import tpu as pltpu, tpu_sc as plsc


def kernel(ids, table):
    batch = ids.shape[0]
    dim = table.shape[1]
    info = plsc.get_sparse_core_info()
    num_cores = info.num_cores
    ns = info.num_subcores
    nw = num_cores * ns
    bpw = batch // nw

    nb = 4
    chunk = bpw // nb

    mesh = plsc.VectorSubcoreMesh(core_axis_name="c", subcore_axis_name="s")
    ids32 = ids.astype(jnp.int32)

    def body(ids_hbm, table_hbm, out_hbm, idx_v, rows_v, shared, isem, gsem, wsem):
        cid = lax.axis_index("c")
        sid = lax.axis_index("s")
        wid = sid * num_cores + cid
        base = wid * bpw
        pltpu.async_copy(ids_hbm.at[pl.ds(base, bpw)], idx_v, isem).wait()
        gathers = [
            pltpu.async_copy(table_hbm.at[idx_v.at[pl.ds(b * chunk, chunk)]],
                             rows_v.at[pl.ds(b * chunk, chunk)], gsem.at[b])
            for b in range(nb)
        ]
        writes = []
        for b in range(nb):
            gathers[b].wait()
            slot = (sid * 2 + (b % 2)) * chunk
            if b >= 2:
                writes[b - 2].wait()
            pltpu.sync_copy(rows_v.at[pl.ds(b * chunk, chunk)],
                            shared.at[pl.ds(slot, chunk)])
            writes.append(pltpu.async_copy(
                shared.at[pl.ds(slot, chunk)],
                out_hbm.at[pl.ds(base + b * chunk, chunk)], wsem.at[b % 2]))
        for w in writes[-2:]:
            w.wait()

    f = pl.kernel(
        body,
        out_type=jax.ShapeDtypeStruct((batch, dim), jnp.float32),
        mesh=mesh,
        scratch_types=[
            pltpu.VMEM((bpw,), jnp.int32),
            pltpu.VMEM((bpw, dim), jnp.float32),
            pltpu.VMEM_SHARED((ns * 2 * chunk, dim), jnp.float32),
            pltpu.SemaphoreType.DMA,
            pltpu.SemaphoreType.DMA((nb,)),
            pltpu.SemaphoreType.DMA((2,)),
        ],
    )
    return f(ids32, table)
